# Initial kernel scaffold; baseline (speedup 1.0000x reference)
import functools

import jax
import jax.numpy as jnp
from jax import lax
from jax.experimental import pallas as pl
from jax.experimental.pallas import tpu as pltpu
from jax.experimental.pallas import tpu_sc as plsc

NTOKEN = 100000
EMB_DIM = 64
BATCH = 4096
SEQ = 50

N = BATCH * SEQ
NC, NS = 2, 16
NW = NC * NS
BPW = N // NW
CHUNK = 128
NCHUNK = BPW // CHUNK


def _embed2(x_flat, W, W_):
    mesh = plsc.VectorSubcoreMesh(core_axis_name="c", subcore_axis_name="s")

    @functools.partial(
        pl.kernel,
        mesh=mesh,
        out_type=jax.ShapeDtypeStruct((N, 2 * EMB_DIM), jnp.float32),
        scratch_types=[
            pltpu.VMEM((NCHUNK, CHUNK), jnp.int32),
            pltpu.VMEM((CHUNK, 2 * EMB_DIM), jnp.float32),
            pltpu.SemaphoreType.DMA,
            pltpu.SemaphoreType.DMA,
            pltpu.SemaphoreType.DMA,
        ],
    )
    def k(x_hbm, w_hbm, w2_hbm, out_hbm, idx_v, comb, sem_a, sem_b, sem_o):
        wid = lax.axis_index("s") * NC + lax.axis_index("c")
        base = wid * BPW
        pltpu.sync_copy(x_hbm.at[wid], idx_v)

        def body(j, carry):
            ca = pltpu.async_copy(
                w_hbm.at[idx_v.at[j]], comb.at[:, pl.ds(0, EMB_DIM)], sem_a)
            cb = pltpu.async_copy(
                w2_hbm.at[idx_v.at[j]], comb.at[:, pl.ds(EMB_DIM, EMB_DIM)],
                sem_b)
            ca.wait()
            cb.wait()
            co = pltpu.async_copy(
                comb, out_hbm.at[pl.ds(base + j * CHUNK, CHUNK)], sem_o)
            co.wait()
            return carry

        lax.fori_loop(0, NCHUNK, body, 0)

    return k(x_flat, W, W_)


def kernel(x, W, W_):
    x_flat = x.reshape(NW, NCHUNK, CHUNK).astype(jnp.int32)
    out = _embed2(x_flat, W, W_)
    return out.reshape(BATCH, SEQ, 2 * EMB_DIM)


# speedup vs baseline: 6.3465x; 6.3465x over previous
import functools

import jax
import jax.numpy as jnp
from jax import lax
from jax.experimental import pallas as pl
from jax.experimental.pallas import tpu as pltpu
from jax.experimental.pallas import tpu_sc as plsc

NTOKEN = 100000
EMB_DIM = 64
BATCH = 4096
SEQ = 50

N = BATCH * SEQ
NC, NS = 2, 16
NW = NC * NS
BPW = N // NW
CHUNK = 128
NCHUNK = BPW // CHUNK
NBUF = 5
ROUNDS = NCHUNK // NBUF


def _embed2(x_flat, Wcat):
    mesh = plsc.VectorSubcoreMesh(core_axis_name="c", subcore_axis_name="s")

    @functools.partial(
        pl.kernel,
        mesh=mesh,
        out_type=jax.ShapeDtypeStruct((N, 2 * EMB_DIM), jnp.float32),
        scratch_types=[
            pltpu.VMEM((NCHUNK, CHUNK), jnp.int32),
            *[pltpu.VMEM((CHUNK, 2 * EMB_DIM), jnp.float32)
              for _ in range(NBUF)],
            *[pltpu.SemaphoreType.DMA for _ in range(2 * NBUF)],
        ],
    )
    def k(x_hbm, w_hbm, out_hbm, idx_v, *rest):
        combs = rest[:NBUF]
        sgs = rest[NBUF:2 * NBUF]
        sos = rest[2 * NBUF:]
        wid = lax.axis_index("s") * NC + lax.axis_index("c")
        base = wid * BPW
        pltpu.sync_copy(x_hbm.at[wid], idx_v)

        def gather(j, b):
            return pltpu.async_copy(w_hbm.at[idx_v.at[j]], combs[b], sgs[b])

        def store(j, b):
            return pltpu.async_copy(
                combs[b], out_hbm.at[pl.ds(base + j * CHUNK, CHUNK)], sos[b])

        def wait_gather(j, b):
            pltpu.make_async_copy(
                w_hbm.at[idx_v.at[j]], combs[b], sgs[b]).wait()

        def wait_store(j, b):
            pltpu.make_async_copy(
                combs[b], out_hbm.at[pl.ds(base + j * CHUNK, CHUNK)],
                sos[b]).wait()

        for b in range(NBUF):
            gather(b, b)

        def outer(r, carry):
            for b in range(NBUF):
                j = r * NBUF + b
                wait_gather(j, b)
                store(j, b)
            for b in range(NBUF):
                j = r * NBUF + b
                wait_store(j, b)
                gather(j + NBUF, b)
            return carry

        lax.fori_loop(0, ROUNDS - 1, outer, 0)

        last = (ROUNDS - 1) * NBUF
        for b in range(NBUF):
            wait_gather(last + b, b)
            store(last + b, b)
        for b in range(NBUF):
            wait_store(last + b, b)

    return k(x_flat, Wcat)


def kernel(x, W, W_):
    Wcat = jnp.concatenate([W, W_], axis=1)
    x_flat = x.reshape(NW, NCHUNK, CHUNK).astype(jnp.int32)
    out = _embed2(x_flat, Wcat)
    return out.reshape(BATCH, SEQ, 2 * EMB_DIM)


# speedup vs baseline: 6.4813x; 1.0212x over previous
import functools

import jax
import jax.numpy as jnp
from jax import lax
from jax.experimental import pallas as pl
from jax.experimental.pallas import tpu as pltpu
from jax.experimental.pallas import tpu_sc as plsc

NTOKEN = 100000
EMB_DIM = 64
BATCH = 4096
SEQ = 50

N = BATCH * SEQ
NC, NS = 2, 16
NW = NC * NS
BPW = N // NW
CHUNK = 128
NCHUNK = BPW // CHUNK
NBUF = 7


def _embed2(x_flat, Wcat):
    mesh = plsc.VectorSubcoreMesh(core_axis_name="c", subcore_axis_name="s")

    @functools.partial(
        pl.kernel,
        mesh=mesh,
        out_type=jax.ShapeDtypeStruct((N, 2 * EMB_DIM), jnp.float32),
        scratch_types=[
            pltpu.VMEM((NCHUNK, CHUNK), jnp.int32),
            *[pltpu.VMEM((CHUNK, 2 * EMB_DIM), jnp.float32)
              for _ in range(NBUF)],
            *[pltpu.SemaphoreType.DMA for _ in range(2 * NBUF)],
        ],
    )
    def k(x_hbm, w_hbm, out_hbm, idx_v, *rest):
        combs = rest[:NBUF]
        sgs = rest[NBUF:2 * NBUF]
        sos = rest[2 * NBUF:]
        wid = lax.axis_index("s") * NC + lax.axis_index("c")
        base = wid * BPW
        pltpu.sync_copy(x_hbm.at[wid], idx_v)

        def gather(j, b):
            return pltpu.async_copy(w_hbm.at[idx_v.at[j]], combs[b], sgs[b])

        def store(j, b):
            return pltpu.async_copy(
                combs[b], out_hbm.at[pl.ds(base + j * CHUNK, CHUNK)], sos[b])

        def wait_gather(j, b):
            pltpu.make_async_copy(
                w_hbm.at[idx_v.at[j]], combs[b], sgs[b]).wait()

        def wait_store(j, b):
            pltpu.make_async_copy(
                combs[b], out_hbm.at[pl.ds(base + j * CHUNK, CHUNK)],
                sos[b]).wait()

        for j in range(NBUF):
            gather(j, j)
        for j in range(NCHUNK):
            b = j % NBUF
            wait_gather(j, b)
            store(j, b)
            if j + NBUF < NCHUNK:
                wait_store(j, b)
                gather(j + NBUF, b)
            else:
                wait_store(j, b)

    return k(x_flat, Wcat)


def kernel(x, W, W_):
    Wcat = jnp.concatenate([W, W_], axis=1)
    x_flat = x.reshape(NW, NCHUNK, CHUNK).astype(jnp.int32)
    out = _embed2(x_flat, Wcat)
    return out.reshape(BATCH, SEQ, 2 * EMB_DIM)
